# Initial kernel scaffold; baseline (speedup 1.0000x reference)
import jax
import jax.numpy as jnp
from jax.experimental import pallas as pl


def kernel(nodes, edge_attr, senders, receivers, W1, b1, W2, b2):
    raise NotImplementedError("write your pallas kernel here")


# speedup vs baseline: 1.4019x; 1.4019x over previous
import functools

import jax
import jax.numpy as jnp
from jax import lax
from jax.experimental import pallas as pl
from jax.experimental.pallas import tpu as pltpu
from jax.experimental.pallas import tpu_sc as plsc

N_NODES = 10000
N_EDGES = 160000
D_FEAT = 256
D_EDGE = 16
D_HID = 1024
D_OUT = 256

NUM_CORES = 2
NUM_SUBCORES = 16
N_WORKERS = NUM_CORES * NUM_SUBCORES
EDGES_PER_WORKER = N_EDGES // N_WORKERS
CHUNK = 40
N_CHUNKS = EDGES_PER_WORKER // CHUNK

@functools.lru_cache(maxsize=None)
def _make_sc_gather():
    mesh = plsc.VectorSubcoreMesh(
        core_axis_name="c", subcore_axis_name="s",
        num_cores=NUM_CORES, num_subcores=NUM_SUBCORES)

    @functools.partial(
        pl.kernel,
        out_type=(
            jax.ShapeDtypeStruct((N_EDGES, D_FEAT), jnp.float32),
            jax.ShapeDtypeStruct((N_EDGES, D_FEAT), jnp.float32),
        ),
        mesh=mesh,
        scratch_types=[
            pltpu.VMEM((CHUNK,), jnp.int32),
            pltpu.VMEM((CHUNK,), jnp.int32),
            pltpu.VMEM((CHUNK, D_FEAT), jnp.float32),
            pltpu.VMEM((CHUNK, D_FEAT), jnp.float32),
            pltpu.SemaphoreType.DMA,
            pltpu.SemaphoreType.DMA,
        ],
    )
    def _sc_gather(nodes_hbm, senders_hbm, receivers_hbm,
                   send_out, recv_out, sidx_v, ridx_v, srows_v, rrows_v,
                   ssem, rsem):
        wid = lax.axis_index("s") * NUM_CORES + lax.axis_index("c")
        base = wid * EDGES_PER_WORKER

        def chunk_body(i, carry):
            off = base + i * CHUNK
            pltpu.sync_copy(senders_hbm.at[pl.ds(off, CHUNK)], sidx_v)
            pltpu.sync_copy(receivers_hbm.at[pl.ds(off, CHUNK)], ridx_v)
            scp = pltpu.async_copy(nodes_hbm.at[sidx_v], srows_v, ssem)
            rcp = pltpu.async_copy(nodes_hbm.at[ridx_v], rrows_v, rsem)
            scp.wait()
            rcp.wait()
            pltpu.sync_copy(srows_v, send_out.at[pl.ds(off, CHUNK)])
            pltpu.sync_copy(rrows_v, recv_out.at[pl.ds(off, CHUNK)])
            return carry

        lax.fori_loop(0, N_CHUNKS, chunk_body, 0)

    return _sc_gather


BE = 1000


def _mlp_body(ea_ref, r_ref, s_ref, w1e_ref, w1r_ref, w1s_ref, b1_ref,
              w2_ref, b2_ref, o_ref):
    acc = jnp.dot(r_ref[...], w1r_ref[...], preferred_element_type=jnp.float32)
    acc = acc + jnp.dot(s_ref[...], w1s_ref[...],
                        preferred_element_type=jnp.float32)
    acc = acc + jnp.dot(ea_ref[...], w1e_ref[...],
                        preferred_element_type=jnp.float32)
    h = jnp.maximum(acc + b1_ref[...], 0.0)
    o_ref[...] = (jnp.dot(h, w2_ref[...], preferred_element_type=jnp.float32)
                  + b2_ref[...])


def _full(shape):
    return pl.BlockSpec(shape, lambda i: (0,) * len(shape))


def _mlp(edge_attr, recv_g, send_g, W1e, W1r, W1s, b1, W2, b2):
    grid = (N_EDGES // BE,)
    return pl.pallas_call(
        _mlp_body,
        grid=grid,
        in_specs=[
            pl.BlockSpec((BE, D_EDGE), lambda i: (i, 0)),
            pl.BlockSpec((BE, D_FEAT), lambda i: (i, 0)),
            pl.BlockSpec((BE, D_FEAT), lambda i: (i, 0)),
            _full((D_EDGE, D_HID)),
            _full((D_FEAT, D_HID)),
            _full((D_FEAT, D_HID)),
            _full((1, D_HID)),
            _full((D_HID, D_OUT)),
            _full((1, D_OUT)),
        ],
        out_specs=pl.BlockSpec((BE, D_OUT), lambda i: (i, 0)),
        out_shape=jax.ShapeDtypeStruct((N_EDGES, D_OUT), jnp.float32),
        compiler_params=pltpu.CompilerParams(
            dimension_semantics=("arbitrary",),
        ),
    )(edge_attr, recv_g, send_g, W1e, W1r, W1s, b1, W2, b2)


def kernel(nodes, edge_attr, senders, receivers, W1, b1, W2, b2):
    send_g, recv_g = _make_sc_gather()(nodes, senders, receivers)
    W1e = W1[:D_EDGE]
    W1r = W1[D_EDGE:D_EDGE + D_FEAT]
    W1s = W1[D_EDGE + D_FEAT:]
    return _mlp(edge_attr, recv_g, send_g, W1e, W1r, W1s,
                b1.reshape(1, -1), W2, b2.reshape(1, -1))


# speedup vs baseline: 1.4422x; 1.0288x over previous
import functools

import jax
import jax.numpy as jnp
from jax import lax
from jax.experimental import pallas as pl
from jax.experimental.pallas import tpu as pltpu
from jax.experimental.pallas import tpu_sc as plsc

N_NODES = 10000
N_EDGES = 160000
D_FEAT = 256
D_EDGE = 16
D_HID = 1024
D_OUT = 256

NUM_CORES = 2
NUM_SUBCORES = 16
N_WORKERS = NUM_CORES * NUM_SUBCORES
EDGES_PER_WORKER = N_EDGES // N_WORKERS
CHUNK = 40
N_CHUNKS = EDGES_PER_WORKER // CHUNK

@functools.lru_cache(maxsize=None)
def _make_sc_gather():
    mesh = plsc.VectorSubcoreMesh(
        core_axis_name="c", subcore_axis_name="s",
        num_cores=NUM_CORES, num_subcores=NUM_SUBCORES)

    @functools.partial(
        pl.kernel,
        out_type=(
            jax.ShapeDtypeStruct((N_EDGES, D_FEAT), jnp.float32),
            jax.ShapeDtypeStruct((N_EDGES, D_FEAT), jnp.float32),
        ),
        mesh=mesh,
        scratch_types=[
            pltpu.VMEM((CHUNK,), jnp.int32),
            pltpu.VMEM((CHUNK,), jnp.int32),
            pltpu.VMEM((CHUNK, D_FEAT), jnp.float32),
            pltpu.VMEM((CHUNK, D_FEAT), jnp.float32),
            pltpu.SemaphoreType.DMA,
            pltpu.SemaphoreType.DMA,
        ],
    )
    def _sc_gather(nodes_hbm, senders_hbm, receivers_hbm,
                   send_out, recv_out, sidx_v, ridx_v, srows_v, rrows_v,
                   ssem, rsem):
        wid = lax.axis_index("s") * NUM_CORES + lax.axis_index("c")
        base = wid * EDGES_PER_WORKER

        def chunk_body(i, carry):
            off = base + i * CHUNK
            pltpu.sync_copy(senders_hbm.at[pl.ds(off, CHUNK)], sidx_v)
            pltpu.sync_copy(receivers_hbm.at[pl.ds(off, CHUNK)], ridx_v)
            scp = pltpu.async_copy(nodes_hbm.at[sidx_v], srows_v, ssem)
            rcp = pltpu.async_copy(nodes_hbm.at[ridx_v], rrows_v, rsem)
            scp.wait()
            rcp.wait()
            pltpu.sync_copy(srows_v, send_out.at[pl.ds(off, CHUNK)])
            pltpu.sync_copy(rrows_v, recv_out.at[pl.ds(off, CHUNK)])
            return carry

        lax.fori_loop(0, N_CHUNKS, chunk_body, 0)

    return _sc_gather


BE = 2000


def _mlp_body(ea_ref, r_ref, s_ref, w1e_ref, w1r_ref, w1s_ref, b1_ref,
              w2_ref, b2_ref, o_ref):
    bf = jnp.bfloat16
    acc = jnp.dot(r_ref[...].astype(bf), w1r_ref[...],
                  preferred_element_type=jnp.float32)
    acc = acc + jnp.dot(s_ref[...].astype(bf), w1s_ref[...],
                        preferred_element_type=jnp.float32)
    acc = acc + jnp.dot(ea_ref[...].astype(bf), w1e_ref[...],
                        preferred_element_type=jnp.float32)
    h = jnp.maximum(acc + b1_ref[...], 0.0).astype(bf)
    o_ref[...] = (jnp.dot(h, w2_ref[...], preferred_element_type=jnp.float32)
                  + b2_ref[...])


def _full(shape):
    return pl.BlockSpec(shape, lambda i: (0,) * len(shape))


def _mlp(edge_attr, recv_g, send_g, W1e, W1r, W1s, b1, W2, b2):
    grid = (N_EDGES // BE,)
    return pl.pallas_call(
        _mlp_body,
        grid=grid,
        in_specs=[
            pl.BlockSpec((BE, D_EDGE), lambda i: (i, 0)),
            pl.BlockSpec((BE, D_FEAT), lambda i: (i, 0)),
            pl.BlockSpec((BE, D_FEAT), lambda i: (i, 0)),
            _full((D_EDGE, D_HID)),
            _full((D_FEAT, D_HID)),
            _full((D_FEAT, D_HID)),
            _full((1, D_HID)),
            _full((D_HID, D_OUT)),
            _full((1, D_OUT)),
        ],
        out_specs=pl.BlockSpec((BE, D_OUT), lambda i: (i, 0)),
        out_shape=jax.ShapeDtypeStruct((N_EDGES, D_OUT), jnp.float32),
        compiler_params=pltpu.CompilerParams(
            dimension_semantics=("arbitrary",),
        ),
    )(edge_attr, recv_g, send_g, W1e, W1r, W1s, b1, W2, b2)


def kernel(nodes, edge_attr, senders, receivers, W1, b1, W2, b2):
    send_g, recv_g = _make_sc_gather()(nodes, senders, receivers)
    W1bf = W1.astype(jnp.bfloat16)
    W1e = W1bf[:D_EDGE]
    W1r = W1bf[D_EDGE:D_EDGE + D_FEAT]
    W1s = W1bf[D_EDGE + D_FEAT:]
    return _mlp(edge_attr, recv_g, send_g, W1e, W1r, W1s,
                b1.reshape(1, -1), W2.astype(jnp.bfloat16),
                b2.reshape(1, -1))


# speedup vs baseline: 1.7149x; 1.1890x over previous
import functools

import jax
import jax.numpy as jnp
from jax import lax
from jax.experimental import pallas as pl
from jax.experimental.pallas import tpu as pltpu
from jax.experimental.pallas import tpu_sc as plsc

N_NODES = 10000
N_EDGES = 160000
D_FEAT = 256
D_EDGE = 16
D_HID = 1024
D_OUT = 256

NUM_CORES = 2
NUM_SUBCORES = 16
N_WORKERS = NUM_CORES * NUM_SUBCORES
N_SLICES = 5
E_SLICE = N_EDGES // N_SLICES
EDGES_PER_WORKER = E_SLICE // N_WORKERS
CHUNK = 40
N_CHUNKS = EDGES_PER_WORKER // CHUNK


@functools.lru_cache(maxsize=None)
def _make_sc_gather():
    mesh = plsc.VectorSubcoreMesh(
        core_axis_name="c", subcore_axis_name="s",
        num_cores=NUM_CORES, num_subcores=NUM_SUBCORES)

    @functools.partial(
        pl.kernel,
        out_type=(
            jax.ShapeDtypeStruct((E_SLICE, D_FEAT), jnp.float32),
            jax.ShapeDtypeStruct((E_SLICE, D_FEAT), jnp.float32),
        ),
        mesh=mesh,
        scratch_types=[
            pltpu.VMEM((CHUNK,), jnp.int32),
            pltpu.VMEM((CHUNK,), jnp.int32),
            pltpu.VMEM((CHUNK, D_FEAT), jnp.float32),
            pltpu.VMEM((CHUNK, D_FEAT), jnp.float32),
            pltpu.SemaphoreType.DMA,
            pltpu.SemaphoreType.DMA,
        ],
    )
    def _sc_gather(nodes_hbm, senders_hbm, receivers_hbm,
                   send_out, recv_out, sidx_v, ridx_v, srows_v, rrows_v,
                   ssem, rsem):
        wid = lax.axis_index("s") * NUM_CORES + lax.axis_index("c")
        base = wid * EDGES_PER_WORKER

        def chunk_body(i, carry):
            off = base + i * CHUNK
            pltpu.sync_copy(senders_hbm.at[pl.ds(off, CHUNK)], sidx_v)
            pltpu.sync_copy(receivers_hbm.at[pl.ds(off, CHUNK)], ridx_v)
            scp = pltpu.async_copy(nodes_hbm.at[sidx_v], srows_v, ssem)
            rcp = pltpu.async_copy(nodes_hbm.at[ridx_v], rrows_v, rsem)
            scp.wait()
            rcp.wait()
            pltpu.sync_copy(srows_v, send_out.at[pl.ds(off, CHUNK)])
            pltpu.sync_copy(rrows_v, recv_out.at[pl.ds(off, CHUNK)])
            return carry

        lax.fori_loop(0, N_CHUNKS, chunk_body, 0)

    return _sc_gather


BE = 2000


def _mlp_body(ea_ref, r_ref, s_ref, w1e_ref, w1r_ref, w1s_ref, b1_ref,
              w2_ref, b2_ref, o_ref):
    bf = jnp.bfloat16
    acc = jnp.dot(r_ref[...].astype(bf), w1r_ref[...],
                  preferred_element_type=jnp.float32)
    acc = acc + jnp.dot(s_ref[...].astype(bf), w1s_ref[...],
                        preferred_element_type=jnp.float32)
    acc = acc + jnp.dot(ea_ref[...].astype(bf), w1e_ref[...],
                        preferred_element_type=jnp.float32)
    h = jnp.maximum(acc + b1_ref[...], 0.0).astype(bf)
    o_ref[...] = (jnp.dot(h, w2_ref[...], preferred_element_type=jnp.float32)
                  + b2_ref[...])


def _full(shape):
    return pl.BlockSpec(shape, lambda i: (0,) * len(shape))


def _mlp(edge_attr, recv_g, send_g, W1e, W1r, W1s, b1, W2, b2):
    n_edges = recv_g.shape[0]
    grid = (n_edges // BE,)
    return pl.pallas_call(
        _mlp_body,
        grid=grid,
        in_specs=[
            pl.BlockSpec((BE, D_EDGE), lambda i: (i, 0)),
            pl.BlockSpec((BE, D_FEAT), lambda i: (i, 0)),
            pl.BlockSpec((BE, D_FEAT), lambda i: (i, 0)),
            _full((D_EDGE, D_HID)),
            _full((D_FEAT, D_HID)),
            _full((D_FEAT, D_HID)),
            _full((1, D_HID)),
            _full((D_HID, D_OUT)),
            _full((1, D_OUT)),
        ],
        out_specs=pl.BlockSpec((BE, D_OUT), lambda i: (i, 0)),
        out_shape=jax.ShapeDtypeStruct((n_edges, D_OUT), jnp.float32),
        compiler_params=pltpu.CompilerParams(
            dimension_semantics=("arbitrary",),
        ),
    )(edge_attr, recv_g, send_g, W1e, W1r, W1s, b1, W2, b2)


def kernel(nodes, edge_attr, senders, receivers, W1, b1, W2, b2):
    gather = _make_sc_gather()
    W1bf = W1.astype(jnp.bfloat16)
    W1e = W1bf[:D_EDGE]
    W1r = W1bf[D_EDGE:D_EDGE + D_FEAT]
    W1s = W1bf[D_EDGE + D_FEAT:]
    W2bf = W2.astype(jnp.bfloat16)
    b1r = b1.reshape(1, -1)
    b2r = b2.reshape(1, -1)

    gathered = []
    for s in range(N_SLICES):
        lo = s * E_SLICE
        send_g, recv_g = gather(nodes, senders[lo:lo + E_SLICE],
                                receivers[lo:lo + E_SLICE])
        gathered.append((send_g, recv_g))
    outs = []
    for s in range(N_SLICES):
        lo = s * E_SLICE
        send_g, recv_g = gathered[s]
        outs.append(_mlp(edge_attr[lo:lo + E_SLICE], recv_g, send_g,
                         W1e, W1r, W1s, b1r, W2bf, b2r))
    return jnp.concatenate(outs, axis=0)
